# baseline (speedup 1.0000x reference)
import functools
import math

import jax
import jax.numpy as jnp
from jax import lax
from jax.experimental import pallas as pl
from jax.experimental.pallas import tpu as pltpu

_H = 12
_LOG2E = math.log2(math.e)


def _mha_fused_kernel(x_ref, wqkv_ref, bqkv_ref, wo_ref, bo_ref, o_ref,
                      *, num_heads, d_k, d_model):
    xb = x_ref[0].astype(jnp.bfloat16)
    S = xb.shape[0]

    def proj(lo, hi):
        return (jnp.dot(xb, wqkv_ref[:, lo:hi],
                        preferred_element_type=jnp.float32)
                + bqkv_ref[:, lo:hi]).astype(jnp.bfloat16)

    q_all = proj(0, d_model)
    k_all = proj(d_model, 2 * d_model)

    ones_pad = jnp.ones((S, 128 - d_k), jnp.bfloat16)

    def qk(h):
        sl = slice(h * d_k, (h + 1) * d_k)
        q_h = q_all[:, sl] * jnp.bfloat16(_LOG2E)
        return lax.dot_general(q_h, k_all[:, sl],
                               (((1,), (1,)), ((), ())),
                               preferred_element_type=jnp.float32
                               ).astype(jnp.bfloat16)

    scores = [None] * num_heads
    scores[0] = qk(0)
    scores[1] = qk(1)
    v_all = proj(2 * d_model, 3 * d_model)
    scores[2] = qk(2)

    parts = []
    for h in range(num_heads):
        if h + 3 < num_heads:
            scores[h + 3] = qk(h + 3)
        s = scores[h]
        m = s.max(axis=-1, keepdims=True)
        p16 = jnp.exp2(s - m)

        v_h = v_all[:, h * d_k:(h + 1) * d_k]
        v_ext = jnp.concatenate([v_h, ones_pad], axis=1)
        pv = jnp.dot(p16, v_ext,
                     preferred_element_type=jnp.float32)
        parts.append(pv[:, :d_k]
                     * pl.reciprocal(pv[:, d_k:], approx=True))
        scores[h] = None

    attn = jnp.concatenate(parts, axis=-1).astype(jnp.bfloat16)
    o_ref[0] = (jnp.dot(attn, wo_ref[...],
                        preferred_element_type=jnp.float32)
                + bo_ref[...])


def _mha_pallas(x, wqkv_b, bqkv_f, wo_b, bo_f):
    B, S, D = x.shape
    kern = functools.partial(_mha_fused_kernel,
                             num_heads=_H, d_k=D // _H, d_model=D)
    return pl.pallas_call(
        kern,
        out_shape=jax.ShapeDtypeStruct((B, S, D), jnp.float32),
        grid=(B,),
        in_specs=[
            pl.BlockSpec((1, S, D), lambda b: (b, 0, 0)),
            pl.BlockSpec((D, 3 * D), lambda b: (0, 0)),
            pl.BlockSpec((1, 3 * D), lambda b: (0, 0)),
            pl.BlockSpec((D, D), lambda b: (0, 0)),
            pl.BlockSpec((1, D), lambda b: (0, 0)),
        ],
        out_specs=pl.BlockSpec((1, S, D), lambda b: (b, 0, 0)),
        compiler_params=pltpu.CompilerParams(
            dimension_semantics=("parallel",),
            vmem_limit_bytes=60 * 1024 * 1024),
    )(x, wqkv_b, bqkv_f, wo_b, bo_f)


def kernel(x, wq, bq, wk, bk, wv, bv, wo, bo, wq_s, bq_s, w_qkv, b_qkv):
    B, S, D = x.shape

    wqkv_b = w_qkv.astype(jnp.bfloat16)
    bqkv_f = b_qkv.reshape(1, 3 * D)
    wo_b = wo.astype(jnp.bfloat16)
    bo_f = bo.reshape(1, D)

    return _mha_pallas(x, wqkv_b, bqkv_f, wo_b, bo_f)


# speedup vs baseline: 1.1049x; 1.1049x over previous
import functools
import math

import jax
import jax.numpy as jnp
from jax import lax
from jax.experimental import pallas as pl
from jax.experimental.pallas import tpu as pltpu

_H = 12
_LOG2E = math.log2(math.e)


def _mha_fused_kernel(x_ref, wqkv_ref, bqkv_ref, wo_ref, bo_ref, o_ref,
                      wqkv16_scr, wo16_scr,
                      *, num_heads, d_k, d_model):
    @pl.when(pl.program_id(1) == 0)
    def _cast_weights():
        wqkv16_scr[...] = wqkv_ref[...].astype(jnp.bfloat16)
        wo16_scr[...] = wo_ref[...].astype(jnp.bfloat16)

    xb = x_ref[0].astype(jnp.bfloat16)
    S = xb.shape[0]

    def proj(lo, hi):
        return (jnp.dot(xb, wqkv16_scr[:, lo:hi],
                        preferred_element_type=jnp.float32)
                + bqkv_ref[:, lo:hi]).astype(jnp.bfloat16)

    q_all = proj(0, d_model)
    k_all = proj(d_model, 2 * d_model)

    ones_pad = jnp.ones((S, 128 - d_k), jnp.bfloat16)

    def qk(h):
        sl = slice(h * d_k, (h + 1) * d_k)
        q_h = q_all[:, sl] * jnp.bfloat16(_LOG2E)
        return lax.dot_general(q_h, k_all[:, sl],
                               (((1,), (1,)), ((), ())),
                               preferred_element_type=jnp.float32
                               ).astype(jnp.bfloat16)

    scores = [None] * num_heads
    scores[0] = qk(0)
    scores[1] = qk(1)
    v_all = proj(2 * d_model, 3 * d_model)

    parts = []
    for h in range(num_heads):
        if h + 2 < num_heads:
            scores[h + 2] = qk(h + 2)
        s = scores[h]
        m = s.max(axis=-1, keepdims=True)
        p16 = jnp.exp2(s - m)

        v_h = v_all[:, h * d_k:(h + 1) * d_k]
        v_ext = jnp.concatenate([v_h, ones_pad], axis=1)
        pv = jnp.dot(p16, v_ext,
                     preferred_element_type=jnp.float32)
        parts.append(pv[:, :d_k]
                     * pl.reciprocal(pv[:, d_k:], approx=True))
        scores[h] = None

    attn = jnp.concatenate(parts, axis=-1).astype(jnp.bfloat16)
    o_ref[0] = (jnp.dot(attn, wo16_scr[...],
                        preferred_element_type=jnp.float32)
                + bo_ref[...])


def _mha_pallas(x, w_qkv, bqkv_f, wo, bo_f):
    B, S, D = x.shape
    n_par = 2 if B % 2 == 0 else 1
    b_seq = B // n_par
    kern = functools.partial(_mha_fused_kernel,
                             num_heads=_H, d_k=D // _H, d_model=D)
    return pl.pallas_call(
        kern,
        out_shape=jax.ShapeDtypeStruct((B, S, D), jnp.float32),
        grid=(n_par, b_seq),
        in_specs=[
            pl.BlockSpec((1, S, D), lambda i, j: (i * b_seq + j, 0, 0)),
            pl.BlockSpec((D, 3 * D), lambda i, j: (0, 0)),
            pl.BlockSpec((1, 3 * D), lambda i, j: (0, 0)),
            pl.BlockSpec((D, D), lambda i, j: (0, 0)),
            pl.BlockSpec((1, D), lambda i, j: (0, 0)),
        ],
        out_specs=pl.BlockSpec((1, S, D), lambda i, j: (i * b_seq + j, 0, 0)),
        scratch_shapes=[
            pltpu.VMEM((D, 3 * D), jnp.bfloat16),
            pltpu.VMEM((D, D), jnp.bfloat16),
        ],
        compiler_params=pltpu.CompilerParams(
            dimension_semantics=("parallel", "arbitrary"),
            vmem_limit_bytes=60 * 1024 * 1024),
    )(x, w_qkv, bqkv_f, wo, bo_f)


def kernel(x, wq, bq, wk, bk, wv, bv, wo, bo, wq_s, bq_s, w_qkv, b_qkv):
    B, S, D = x.shape
    return _mha_pallas(x, w_qkv, b_qkv.reshape(1, 3 * D),
                       wo, bo.reshape(1, D))


# speedup vs baseline: 1.1296x; 1.0224x over previous
import functools
import math

import jax
import jax.numpy as jnp
from jax import lax
from jax.experimental import pallas as pl
from jax.experimental.pallas import tpu as pltpu

_H = 12
_LOG2E = math.log2(math.e)


def _mha_fused_kernel(x_ref, wqkv_ref, bqkv_ref, wo_ref, bo_ref, o_ref,
                      wqkv16_scr, wo16_scr,
                      *, num_heads, d_k, d_model):
    @pl.when(pl.program_id(1) == 0)
    def _cast_weights():
        wqkv16_scr[...] = wqkv_ref[...].astype(jnp.bfloat16)
        wo16_scr[...] = wo_ref[...].astype(jnp.bfloat16)

    xb = x_ref[0].astype(jnp.bfloat16)
    S = xb.shape[0]

    def proj(lo, hi):
        return (jnp.dot(xb, wqkv16_scr[:, lo:hi],
                        preferred_element_type=jnp.float32)
                + bqkv_ref[lo:hi]).astype(jnp.bfloat16)

    q_all = proj(0, d_model)
    k_all = proj(d_model, 2 * d_model)

    ones_pad = jnp.ones((S, 128 - d_k), jnp.bfloat16)

    def qk(h):
        sl = slice(h * d_k, (h + 1) * d_k)
        q_h = q_all[:, sl] * jnp.bfloat16(_LOG2E)
        return lax.dot_general(q_h, k_all[:, sl],
                               (((1,), (1,)), ((), ())),
                               preferred_element_type=jnp.float32
                               ).astype(jnp.bfloat16)

    scores = [None] * num_heads
    scores[0] = qk(0)
    scores[1] = qk(1)
    v_all = proj(2 * d_model, 3 * d_model)

    parts = []
    for h in range(num_heads):
        if h + 2 < num_heads:
            scores[h + 2] = qk(h + 2)
        s = scores[h]
        m = s.max(axis=-1, keepdims=True)
        p16 = jnp.exp2(s - m)

        v_h = v_all[:, h * d_k:(h + 1) * d_k]
        v_ext = jnp.concatenate([v_h, ones_pad], axis=1)
        pv = jnp.dot(p16, v_ext,
                     preferred_element_type=jnp.float32)
        parts.append(pv[:, :d_k]
                     * pl.reciprocal(pv[:, d_k:], approx=True))
        scores[h] = None

    attn = jnp.concatenate(parts, axis=-1).astype(jnp.bfloat16)
    o_ref[0] = (jnp.dot(attn, wo16_scr[...],
                        preferred_element_type=jnp.float32)
                + bo_ref[...])


def _mha_pallas(x, w_qkv, bqkv_f, wo, bo_f):
    B, S, D = x.shape
    n_par = 2 if B % 2 == 0 else 1
    b_seq = B // n_par
    kern = functools.partial(_mha_fused_kernel,
                             num_heads=_H, d_k=D // _H, d_model=D)
    return pl.pallas_call(
        kern,
        out_shape=jax.ShapeDtypeStruct((B, S, D), jnp.float32),
        grid=(n_par, b_seq),
        in_specs=[
            pl.BlockSpec((1, S, D), lambda i, j: (i * b_seq + j, 0, 0)),
            pl.BlockSpec((D, 3 * D), lambda i, j: (0, 0)),
            pl.BlockSpec((3 * D,), lambda i, j: (0,)),
            pl.BlockSpec((D, D), lambda i, j: (0, 0)),
            pl.BlockSpec((D,), lambda i, j: (0,)),
        ],
        out_specs=pl.BlockSpec((1, S, D), lambda i, j: (i * b_seq + j, 0, 0)),
        scratch_shapes=[
            pltpu.VMEM((D, 3 * D), jnp.bfloat16),
            pltpu.VMEM((D, D), jnp.bfloat16),
        ],
        compiler_params=pltpu.CompilerParams(
            dimension_semantics=("parallel", "arbitrary"),
            vmem_limit_bytes=60 * 1024 * 1024),
    )(x, w_qkv, bqkv_f, wo, bo_f)


def kernel(x, wq, bq, wk, bk, wv, bv, wo, bo, wq_s, bq_s, w_qkv, b_qkv):
    return _mha_pallas(x, w_qkv, b_qkv, wo, bo)


# speedup vs baseline: 1.1442x; 1.0129x over previous
import functools
import math

import jax
import jax.numpy as jnp
from jax import lax
from jax.experimental import pallas as pl
from jax.experimental.pallas import tpu as pltpu

_H = 12
_LOG2E = math.log2(math.e)


def _mha_fused_kernel(x_ref, wqkv_ref, bqkv_ref, wo_ref, bo_ref, o_ref,
                      wqkv16_scr, wo16_scr,
                      *, num_heads, d_k, d_model):
    @pl.when(pl.program_id(1) == 0)
    def _cast_weights():
        wqkv16_scr[...] = wqkv_ref[...].astype(jnp.bfloat16)
        wo16_scr[...] = wo_ref[...].astype(jnp.bfloat16)

    xb = x_ref[0].astype(jnp.bfloat16)
    S = xb.shape[0]

    def proj(lo, hi):
        return (jnp.dot(xb, wqkv16_scr[:, lo:hi],
                        preferred_element_type=jnp.float32)
                + bqkv_ref[lo:hi]).astype(jnp.bfloat16)

    q_all = proj(0, d_model)
    k_all = proj(d_model, 2 * d_model)

    ones_pad = jnp.ones((S, 128 - d_k), jnp.bfloat16)

    def qk(h):
        sl = slice(h * d_k, (h + 1) * d_k)
        q_h = q_all[:, sl] * jnp.bfloat16(_LOG2E)
        return lax.dot_general(q_h, k_all[:, sl],
                               (((1,), (1,)), ((), ())),
                               preferred_element_type=jnp.float32
                               ).astype(jnp.bfloat16)

    scores = [None] * num_heads
    scores[0] = qk(0)
    scores[1] = qk(1)
    v_all = proj(2 * d_model, 3 * d_model)
    v_ext_all = jnp.concatenate(
        [blk for h in range(num_heads)
         for blk in (v_all[:, h * d_k:(h + 1) * d_k], ones_pad)], axis=1)

    parts = []
    for h in range(num_heads):
        if h + 2 < num_heads:
            scores[h + 2] = qk(h + 2)
        s = scores[h]
        m = s.max(axis=-1, keepdims=True)
        p16 = jnp.exp2(s - m)

        pv = jnp.dot(p16, v_ext_all[:, h * 128:(h + 1) * 128],
                     preferred_element_type=jnp.float32)
        parts.append(pv[:, :d_k]
                     * pl.reciprocal(pv[:, d_k:], approx=True))
        scores[h] = None

    attn = jnp.concatenate(parts, axis=-1).astype(jnp.bfloat16)
    o_ref[0] = (jnp.dot(attn, wo16_scr[...],
                        preferred_element_type=jnp.float32)
                + bo_ref[...])


def _mha_pallas(x, w_qkv, bqkv_f, wo, bo_f):
    B, S, D = x.shape
    n_par = 2 if B % 2 == 0 else 1
    b_seq = B // n_par
    kern = functools.partial(_mha_fused_kernel,
                             num_heads=_H, d_k=D // _H, d_model=D)
    return pl.pallas_call(
        kern,
        out_shape=jax.ShapeDtypeStruct((B, S, D), jnp.float32),
        grid=(n_par, b_seq),
        in_specs=[
            pl.BlockSpec((1, S, D), lambda i, j: (i * b_seq + j, 0, 0)),
            pl.BlockSpec((D, 3 * D), lambda i, j: (0, 0)),
            pl.BlockSpec((3 * D,), lambda i, j: (0,)),
            pl.BlockSpec((D, D), lambda i, j: (0, 0)),
            pl.BlockSpec((D,), lambda i, j: (0,)),
        ],
        out_specs=pl.BlockSpec((1, S, D), lambda i, j: (i * b_seq + j, 0, 0)),
        scratch_shapes=[
            pltpu.VMEM((D, 3 * D), jnp.bfloat16),
            pltpu.VMEM((D, D), jnp.bfloat16),
        ],
        compiler_params=pltpu.CompilerParams(
            dimension_semantics=("parallel", "arbitrary"),
            vmem_limit_bytes=60 * 1024 * 1024),
    )(x, w_qkv, bqkv_f, wo, bo_f)


def kernel(x, wq, bq, wk, bk, wv, bv, wo, bo, wq_s, bq_s, w_qkv, b_qkv):
    return _mha_pallas(x, w_qkv, b_qkv, wo, bo)
